# baseline (speedup 1.0000x reference)
import jax
import jax.numpy as jnp
from jax.experimental import pallas as pl
from jax.experimental.pallas import tpu as pltpu

_EPS = 1e-9
_R = 8
_G = 32


def _diversity_kernel(tr_ref, out_ref, *, T, scale):
    b = pl.program_id(0)
    nsteps = pl.num_programs(0)
    v = tr_ref[...]
    xf = v[0]
    yf = v[1]
    G, nc, W = xf.shape
    acc1 = jnp.zeros((G, nc, W), jnp.float32)
    acc2 = jnp.zeros((G, nc, W), jnp.float32)
    half = _R // 2
    for lc in range(half + 1):
        if lc == 0:
            xl, yl = xf, yf
            rcs = [(1, 2), (2, 2), (3, 2), (4, 1)]
        else:
            xl = pltpu.roll(xf, (_R - lc) * T, axis=2)
            yl = pltpu.roll(yf, (_R - lc) * T, axis=2)
            if lc == half:
                rcs = [(0, 1), (1, 2), (2, 2), (3, 2), (4, 1)]
            else:
                rcs = [(rc, 2) for rc in range(nc)]
        for rc, w in rcs:
            if rc == 0:
                xr, yr = xl, yl
            else:
                xr = pltpu.roll(xl, nc - rc, axis=1)
                yr = pltpu.roll(yl, nc - rc, axis=1)
            dx = xf - xr
            dy = yf - yr
            s2 = dx * dx + dy * dy + _EPS
            d = s2 * jax.lax.rsqrt(s2)
            if w == 1:
                acc1 = acc1 + d
            else:
                acc2 = acc2 + d
    s = 2.0 * jnp.sum(acc2) + jnp.sum(acc1)
    out_ref[:, :, :] = jnp.broadcast_to(s, (1, 1, 1))


def kernel(predicted_trajectory):
    B, C, T, _ = predicted_trajectory.shape
    nc = C // _R
    W = _R * T
    tp = jnp.moveaxis(predicted_trajectory, 3, 0).reshape(2, B, nc, W)
    import functools
    scale = 1.0 / (T * B * C * (C - 1))
    out = pl.pallas_call(
        functools.partial(_diversity_kernel, T=T, scale=scale),
        grid=(B // _G,),
        in_specs=[pl.BlockSpec((2, _G, nc, W), lambda b: (0, b, 0, 0))],
        out_specs=pl.BlockSpec((1, 1, 1), lambda b: (b, 0, 0)),
        out_shape=jax.ShapeDtypeStruct((B // _G, 1, 1), jnp.float32),
        compiler_params=pltpu.CompilerParams(
            dimension_semantics=("parallel",)),
    )(tp)
    return -jnp.sum(out) * scale


# speedup vs baseline: 1.0925x; 1.0925x over previous
import jax
import jax.numpy as jnp
from jax.experimental import pallas as pl
from jax.experimental.pallas import tpu as pltpu

_EPS = 1e-9
_R = 8
_G = 64


def _diversity_kernel(tr_ref, out_ref, *, T, scale):
    b = pl.program_id(0)
    nsteps = pl.num_programs(0)
    v = tr_ref[...]
    xf = v[0]
    yf = v[1]
    G, nc, W = xf.shape
    acc1 = jnp.zeros((G, nc, W), jnp.float32)
    acc2 = jnp.zeros((G, nc, W), jnp.float32)
    half = _R // 2
    for lc in range(half + 1):
        if lc == 0:
            xl, yl = xf, yf
            rcs = [(1, 2), (2, 2), (3, 2), (4, 1)]
        else:
            xl = pltpu.roll(xf, (_R - lc) * T, axis=2)
            yl = pltpu.roll(yf, (_R - lc) * T, axis=2)
            if lc == half:
                rcs = [(0, 1), (1, 2), (2, 2), (3, 2), (4, 1)]
            else:
                rcs = [(rc, 2) for rc in range(nc)]
        for rc, w in rcs:
            if rc == 0:
                xr, yr = xl, yl
            else:
                xr = pltpu.roll(xl, nc - rc, axis=1)
                yr = pltpu.roll(yl, nc - rc, axis=1)
            dx = xf - xr
            dy = yf - yr
            s2 = dx * dx + dy * dy + _EPS
            d = s2 * jax.lax.rsqrt(s2)
            if w == 1:
                acc1 = acc1 + d
            else:
                acc2 = acc2 + d
    s = 2.0 * jnp.sum(acc2) + jnp.sum(acc1)

    @pl.when(b == 0)
    def _():
        out_ref[:, :] = jnp.zeros_like(out_ref)

    out_ref[:, :] = out_ref[:, :] + s

    @pl.when(b == nsteps - 1)
    def _():
        out_ref[:, :] = out_ref[:, :] * (-scale)


def kernel(predicted_trajectory):
    B, C, T, _ = predicted_trajectory.shape
    nc = C // _R
    W = _R * T
    tp = jnp.moveaxis(predicted_trajectory, 3, 0).reshape(2, B, nc, W)
    import functools
    scale = 1.0 / (T * B * C * (C - 1))
    out = pl.pallas_call(
        functools.partial(_diversity_kernel, T=T, scale=scale),
        grid=(B // _G,),
        in_specs=[pl.BlockSpec((2, _G, nc, W), lambda b: (0, b, 0, 0))],
        out_specs=pl.BlockSpec((1, 1), lambda b: (0, 0)),
        out_shape=jax.ShapeDtypeStruct((1, 1), jnp.float32),
    )(tp)
    return out[0, 0]


# speedup vs baseline: 1.1076x; 1.0138x over previous
import functools

import jax
import jax.numpy as jnp
from jax.experimental import pallas as pl
from jax.experimental.pallas import tpu as pltpu

_EPS = 1e-9
_R = 8
_G = 32


def _diversity_kernel(tr_ref, out_ref, *, T, scale):
    b = pl.program_id(0)
    nsteps = pl.num_programs(0)
    v = tr_ref[...]
    xf = v[0]
    yf = v[1]
    G, nc, W = xf.shape
    acc1 = jnp.zeros((G, nc, W), jnp.float32)
    acc2 = jnp.zeros((G, nc, W), jnp.float32)
    half = _R // 2
    for lc in range(half + 1):
        if lc == 0:
            xl, yl = xf, yf
            rcs = [(1, 2), (2, 2), (3, 2), (4, 1)]
        else:
            xl = pltpu.roll(xf, (_R - lc) * T, axis=2)
            yl = pltpu.roll(yf, (_R - lc) * T, axis=2)
            if lc == half:
                rcs = [(0, 1), (1, 2), (2, 2), (3, 2), (4, 1)]
            else:
                rcs = [(rc, 2) for rc in range(nc)]
        for rc, w in rcs:
            if rc == 0:
                xr, yr = xl, yl
            else:
                xr = pltpu.roll(xl, nc - rc, axis=1)
                yr = pltpu.roll(yl, nc - rc, axis=1)
            dx = xf - xr
            dy = yf - yr
            s2 = dx * dx + dy * dy + _EPS
            d = s2 * jax.lax.rsqrt(s2)
            if w == 1:
                acc1 = acc1 + d
            else:
                acc2 = acc2 + d
    s = 2.0 * jnp.sum(acc2) + jnp.sum(acc1)

    @pl.when(b == 0)
    def _():
        out_ref[:, :] = jnp.zeros_like(out_ref)

    out_ref[:, :] = out_ref[:, :] + s

    @pl.when(b == nsteps - 1)
    def _():
        out_ref[:, :] = out_ref[:, :] * (-scale)


def kernel(predicted_trajectory):
    B, C, T, _ = predicted_trajectory.shape
    nc = C // _R
    W = _R * T
    tp = jnp.moveaxis(predicted_trajectory, 3, 0).reshape(2, B, nc, W)
    scale = 1.0 / (T * B * C * (C - 1))
    out = pl.pallas_call(
        functools.partial(_diversity_kernel, T=T, scale=scale),
        grid=(B // _G,),
        in_specs=[pl.BlockSpec((2, _G, nc, W), lambda b: (0, b, 0, 0))],
        out_specs=pl.BlockSpec((1, 1), lambda b: (0, 0)),
        out_shape=jax.ShapeDtypeStruct((1, 1), jnp.float32),
    )(tp)
    return out[0, 0]
